# baseline (speedup 1.0000x reference)
import functools

import jax
import jax.numpy as jnp
from jax import lax
from jax.experimental import pallas as pl
from jax.experimental.pallas import tpu as pltpu
from jax.experimental.pallas import tpu_sc as plsc

NUM_CLASSES = 1000000
LAT_DIM = 32
BATCH = 16384

_INFO = plsc.get_sparse_core_info()
_NC, _NS, _L = _INFO.num_cores, _INFO.num_subcores, _INFO.num_lanes
_NW = _NC * _NS
_BPW = BATCH // _NW
_CHUNK = 128
_NCHUNK = _BPW // _CHUNK


def _body(idx_hbm, tab2_hbm, mu_hbm, sig_hbm,
          idx_raw, idx_mu, idx_sig, mu_v, sig_v, sem):
    wid = lax.axis_index("s") * _NC + lax.axis_index("c")
    base = wid * _BPW

    pltpu.sync_copy(idx_hbm.at[pl.ds(base, _BPW)], idx_raw)

    for k in range(_BPW // _L):
        j, o = k // (_CHUNK // _L), (k % (_CHUNK // _L)) * _L
        v = idx_raw[pl.ds(k * _L, _L)]
        two = v + v
        idx_mu[j, pl.ds(o, _L)] = two
        idx_sig[j, pl.ds(o, _L)] = two + 1

    copies = []
    for j in range(_NCHUNK):
        copies.append(pltpu.async_copy(
            tab2_hbm.at[idx_mu.at[j]], mu_v.at[pl.ds(j * _CHUNK, _CHUNK)], sem))
        copies.append(pltpu.async_copy(
            tab2_hbm.at[idx_sig.at[j]], sig_v.at[pl.ds(j * _CHUNK, _CHUNK)], sem))
    for c in copies:
        c.wait()

    def exp_rows(i, _):
        r0 = i * 4
        for r in range(4):
            sig_v[r0 + r, 0:_L] = jnp.exp(sig_v[r0 + r, 0:_L])
            sig_v[r0 + r, _L:2 * _L] = jnp.exp(sig_v[r0 + r, _L:2 * _L])
        return _

    lax.fori_loop(0, _BPW // 4, exp_rows, None)

    pltpu.sync_copy(mu_v, mu_hbm.at[pl.ds(base, _BPW)])
    pltpu.sync_copy(sig_v, sig_hbm.at[pl.ds(base, _BPW)])


@jax.jit
def _prior_sc(indices, table2):
    f32 = jnp.float32
    run = functools.partial(
        pl.kernel,
        out_type=(jax.ShapeDtypeStruct((BATCH, LAT_DIM), f32),
                  jax.ShapeDtypeStruct((BATCH, LAT_DIM), f32)),
        mesh=plsc.VectorSubcoreMesh(core_axis_name="c", subcore_axis_name="s"),
        compiler_params=pltpu.CompilerParams(use_tc_tiling_on_sc=False),
        scratch_types=[
            pltpu.VMEM((_BPW,), jnp.int32),
            pltpu.VMEM((_NCHUNK, _CHUNK), jnp.int32),
            pltpu.VMEM((_NCHUNK, _CHUNK), jnp.int32),
            pltpu.VMEM((_BPW, LAT_DIM), f32),
            pltpu.VMEM((_BPW, LAT_DIM), f32),
            pltpu.SemaphoreType.DMA,
        ],
    )(_body)
    return run(indices, table2)


def kernel(indices, table):
    table2 = table.reshape(2 * NUM_CLASSES, LAT_DIM)
    mu, sigma = _prior_sc(indices.astype(jnp.int32), table2)
    return (mu, sigma)
